# baseline (speedup 1.0000x reference)
import functools

import jax
import jax.numpy as jnp
from jax import lax
from jax.experimental import pallas as pl
from jax.experimental.pallas import tpu as pltpu
from jax.experimental.pallas import tpu_sc as plsc


def _build_gather(B, VB, D, dtype):
    info = plsc.get_sparse_core_info()
    NW = info.num_cores * info.num_subcores
    b_per_w = B // NW
    OC = 32
    G = 8

    mesh = plsc.VectorSubcoreMesh(core_axis_name="c", subcore_axis_name="s")

    @functools.partial(
        pl.kernel,
        mesh=mesh,
        out_type=jax.ShapeDtypeStruct((B, D), dtype),
        scratch_types=[
            pltpu.VMEM((b_per_w,), jnp.int32),
            pltpu.VMEM((G, 8, D), dtype),
            pltpu.VMEM((OC, D), dtype),
            pltpu.SemaphoreType.DMA,
            pltpu.SemaphoreType.DMA,
        ],
    )
    def gather_kernel(idx_hbm, table_hbm, out_hbm, idx_v, blocks_v, out_v,
                      sem, sem2):
        wid = lax.axis_index("s") * info.num_cores + lax.axis_index("c")
        base = wid * b_per_w
        pltpu.sync_copy(idx_hbm.at[pl.ds(base, b_per_w)], idx_v)

        def do_chunk(oc, _):
            for h in range(OC // 16):
                vec = idx_v[pl.ds(oc * OC + h * 16, 16)]
                blk = jnp.right_shift(vec, 3)
                sub = jnp.bitwise_and(vec, 7)
                for half in range(2):
                    for k in range(G):
                        lane = half * G + k
                        pltpu.async_copy(
                            table_hbm.at[blk[lane]], blocks_v.at[k], sem
                        )
                    pltpu.make_async_copy(
                        table_hbm.at[pl.ds(0, G)], blocks_v, sem
                    ).wait()
                    for k in range(G):
                        lane = half * G + k
                        s = sub[lane]
                        orow = h * 16 + half * G + k

                        def copy_col(cc, _, k=k, s=s, orow=orow):
                            out_v[orow, pl.ds(cc * 16, 16)] = (
                                blocks_v[k, s, pl.ds(cc * 16, 16)]
                            )
                            return ()

                        lax.fori_loop(0, D // 16, copy_col, (), unroll=8)
                        if D % 16:
                            out_v[orow, pl.ds(D - 16, 16)] = (
                                blocks_v[k, s, pl.ds(D - 16, 16)]
                            )
            pltpu.sync_copy(
                out_v, out_hbm.at[pl.ds(base + oc * OC, OC)]
            )
            return ()

        lax.fori_loop(0, b_per_w // OC, do_chunk, (), unroll=False)

    return gather_kernel


def kernel(x, index, logits_table):
    B = index.shape[0]
    V, D = logits_table.shape
    table3 = logits_table.reshape(V // 8, 8, D)
    gather = _build_gather(B, V // 8, D, logits_table.dtype)
    return gather(index, table3)


# speedup vs baseline: 3.7366x; 3.7366x over previous
import functools

import jax
import jax.numpy as jnp
from jax import lax
from jax.experimental import pallas as pl
from jax.experimental.pallas import tpu as pltpu
from jax.experimental.pallas import tpu_sc as plsc


def _build_gather(B, VB, D, dtype):
    info = plsc.get_sparse_core_info()
    NW = info.num_cores * info.num_subcores
    b_per_w = B // NW
    OC = 32
    G = 8

    mesh = plsc.VectorSubcoreMesh(core_axis_name="c", subcore_axis_name="s")

    @functools.partial(
        pl.kernel,
        mesh=mesh,
        out_type=jax.ShapeDtypeStruct((B, D), dtype),
        scratch_types=[
            pltpu.VMEM((b_per_w,), jnp.int32),
            pltpu.VMEM((G, 8, D), dtype),
            pltpu.VMEM((OC, D), dtype),
            pltpu.SemaphoreType.DMA,
            pltpu.SemaphoreType.DMA,
        ],
    )
    def gather_kernel(idx_hbm, table2_hbm, out_hbm, idx_v, blocks_v, out_v,
                      sem, sem2):
        table_hbm = table2_hbm.reshape(VB, 8, D)
        wid = lax.axis_index("s") * info.num_cores + lax.axis_index("c")
        base = wid * b_per_w
        pltpu.sync_copy(idx_hbm.at[pl.ds(base, b_per_w)], idx_v)

        def do_chunk(oc, _):
            for h in range(OC // 16):
                vec = idx_v[pl.ds(oc * OC + h * 16, 16)]
                blk = jnp.right_shift(vec, 3)
                sub = jnp.bitwise_and(vec, 7)
                for half in range(2):
                    for k in range(G):
                        lane = half * G + k
                        pltpu.async_copy(
                            table_hbm.at[blk[lane]], blocks_v.at[k], sem
                        )
                    pltpu.make_async_copy(
                        table_hbm.at[pl.ds(0, G)], blocks_v, sem
                    ).wait()
                    for k in range(G):
                        lane = half * G + k
                        s = sub[lane]
                        orow = h * 16 + half * G + k

                        def copy_col(cc, _, k=k, s=s, orow=orow):
                            out_v[orow, pl.ds(cc * 16, 16)] = (
                                blocks_v[k, s, pl.ds(cc * 16, 16)]
                            )
                            return ()

                        lax.fori_loop(0, D // 16, copy_col, (), unroll=8)
                        if D % 16:
                            out_v[orow, pl.ds(D - 16, 16)] = (
                                blocks_v[k, s, pl.ds(D - 16, 16)]
                            )
            pltpu.sync_copy(
                out_v, out_hbm.at[pl.ds(base + oc * OC, OC)]
            )
            return ()

        lax.fori_loop(0, b_per_w // OC, do_chunk, (), unroll=False)

    return gather_kernel


def kernel(x, index, logits_table):
    B = index.shape[0]
    V, D = logits_table.shape
    gather = _build_gather(B, V // 8, D, logits_table.dtype)
    return gather(index, logits_table)


# speedup vs baseline: 3.9580x; 1.0592x over previous
import functools

import jax
import jax.numpy as jnp
from jax import lax
from jax.experimental import pallas as pl
from jax.experimental.pallas import tpu as pltpu
from jax.experimental.pallas import tpu_sc as plsc


def _build_tail_repack(V, D, DM, dtype):
    DT = D - DM
    ROWS = 2000
    grid = V // ROWS

    def repack_kernel(t_ref, o_ref):
        o_ref[:, :DT] = t_ref[:, :DT]
        o_ref[:, DT:] = jnp.zeros((ROWS, 128 - DT), dtype)

    return pl.pallas_call(
        repack_kernel,
        grid=(grid,),
        in_specs=[
            pl.BlockSpec((ROWS, 128), lambda i: (i, DM // 128)),
        ],
        out_specs=pl.BlockSpec((ROWS, 128), lambda i: (i, 0)),
        out_shape=jax.ShapeDtypeStruct((V, 128), dtype),
    )


def _build_gather(B, V, D, dtype):
    info = plsc.get_sparse_core_info()
    NW = info.num_cores * info.num_subcores
    b_per_w = B // NW
    C = 64
    DM = (D // 128) * 128
    DT = D - DM

    mesh = plsc.VectorSubcoreMesh(core_axis_name="c", subcore_axis_name="s")

    @functools.partial(
        pl.kernel,
        mesh=mesh,
        out_type=jax.ShapeDtypeStruct((B, D), dtype),
        scratch_types=[
            pltpu.VMEM((b_per_w,), jnp.int32),
            pltpu.VMEM((C, D), dtype),
            pltpu.VMEM((C, 128), dtype),
            pltpu.SemaphoreType.DMA,
        ],
    )
    def gather_kernel(idx_hbm, table_hbm, tail_hbm, out_hbm,
                      idx_v, out_v, tail_v, sem):
        wid = lax.axis_index("s") * info.num_cores + lax.axis_index("c")
        base = wid * b_per_w
        pltpu.sync_copy(idx_hbm.at[pl.ds(base, b_per_w)], idx_v)

        def do_chunk(c, _):
            cbase = c * C
            idx_c = idx_v.at[pl.ds(cbase, C)]
            pltpu.async_copy(
                table_hbm.at[idx_c, pl.ds(0, DM)],
                out_v.at[:, pl.ds(0, DM)], sem
            )
            pltpu.async_copy(tail_hbm.at[idx_c], tail_v, sem)
            pltpu.make_async_copy(
                table_hbm.at[pl.ds(0, C), pl.ds(0, DM)],
                out_v.at[:, pl.ds(0, DM)], sem
            ).wait()
            pltpu.make_async_copy(
                tail_hbm.at[pl.ds(0, C)], tail_v, sem
            ).wait()

            def stitch(r, _):
                for t in range(DT // 16):
                    out_v[r, pl.ds(DM + t * 16, 16)] = (
                        tail_v[r, pl.ds(t * 16, 16)]
                    )
                if DT % 16:
                    out_v[r, pl.ds(D - 16, 16)] = tail_v[r, pl.ds(DT - 16, 16)]
                return ()

            lax.fori_loop(0, C, stitch, (), unroll=4)
            pltpu.sync_copy(out_v, out_hbm.at[pl.ds(base + cbase, C)])
            return ()

        lax.fori_loop(0, b_per_w // C, do_chunk, (), unroll=False)

    return gather_kernel


def kernel(x, index, logits_table):
    B = index.shape[0]
    V, D = logits_table.shape
    DM = (D // 128) * 128
    tail_pad = _build_tail_repack(V, D, DM, logits_table.dtype)(logits_table)
    gather = _build_gather(B, V, D, logits_table.dtype)
    return gather(index, logits_table, tail_pad)
